# baseline (speedup 1.0000x reference)
import functools

import jax
import jax.numpy as jnp
from jax import lax
from jax.experimental import pallas as pl
from jax.experimental.pallas import tpu as pltpu
from jax.experimental.pallas import tpu_sc as plsc

VOCAB_SIZE = 1000000
HIDDEN = 128

NC = 2
NS = 16
NW = NC * NS

CHUNK = 128
B_TOTAL = 4096 * 200
B_PER_W = B_TOTAL // NW
NCHUNK = B_PER_W // CHUNK


def _mesh():
    return plsc.VectorSubcoreMesh(
        core_axis_name="c", subcore_axis_name="s", num_cores=NC, num_subcores=NS
    )


NBUF = 4


@functools.partial(
    pl.kernel,
    out_type=jax.ShapeDtypeStruct((NW, NCHUNK, CHUNK, HIDDEN), jnp.float32),
    mesh=_mesh(),
    scratch_types=[
        pltpu.VMEM((NCHUNK, CHUNK), jnp.int32),
        pltpu.VMEM((NBUF, CHUNK, HIDDEN), jnp.float32),
    ]
    + [pltpu.SemaphoreType.DMA] * (2 * NBUF),
)
def _gather_kernel(idx_hbm, table_hbm, out_hbm, idx_v, rows, *sems):
    gsems, wsems = sems[:NBUF], sems[NBUF:]
    wid = lax.axis_index("s") * NC + lax.axis_index("c")

    pltpu.sync_copy(idx_hbm.at[wid], idx_v)

    def gstart(j, b):
        pltpu.make_async_copy(table_hbm.at[idx_v.at[j]], rows.at[b], gsems[b]).start()

    def gwait(j, b):
        pltpu.make_async_copy(table_hbm.at[idx_v.at[j]], rows.at[b], gsems[b]).wait()

    def wstart(j, b):
        pltpu.make_async_copy(rows.at[b], out_hbm.at[wid, j], wsems[b]).start()

    def wwait(j, b):
        pltpu.make_async_copy(rows.at[b], out_hbm.at[wid, j], wsems[b]).wait()

    gstart(0, 0)
    gstart(1, 1)

    def quad(p, _):
        for b in range(NBUF):
            j = NBUF * p + b
            bn = (b + 2) % NBUF

            @pl.when(j >= 2)
            def _():
                wwait(j - 2, bn)

            @pl.when(j + 2 < NCHUNK)
            def _():
                gstart(j + 2, bn)

            gwait(j, b)
            wstart(j, b)
        return 0

    lax.fori_loop(0, NCHUNK // NBUF, quad, 0)
    wwait(NCHUNK - 2, (NCHUNK - 2) % NBUF)
    wwait(NCHUNK - 1, (NCHUNK - 1) % NBUF)


def kernel(inputs, weight):
    idx = inputs.astype(jnp.int32).reshape(NW, NCHUNK, CHUNK)
    out = _gather_kernel(idx, weight)
    return out.reshape(4096, 200, HIDDEN)
